# baseline (speedup 1.0000x reference)
import functools

import jax
import jax.numpy as jnp
from jax import lax
from jax.experimental import pallas as pl
from jax.experimental.pallas import tpu as pltpu
from jax.experimental.pallas import tpu_sc as plsc

NUM_CLASSES = 100000
FEAT_DIM = 128
BATCH = 16384

NC = 2
NS = 16
LANES = 16
NW = NC * NS
BPW = BATCH // NW
CHUNK = 128
NCHUNK = BPW // CHUNK
NGROUP = CHUNK // LANES

_sc_mesh = plsc.VectorSubcoreMesh(core_axis_name="c", subcore_axis_name="s")


def _rsqrt_newton(x):
    xi = lax.bitcast_convert_type(x, jnp.int32)
    yi = jnp.int32(0x5F3759DF) - (xi >> 1)
    y = lax.bitcast_convert_type(yi, jnp.float32)
    for _ in range(3):
        y = y * (jnp.float32(1.5) - jnp.float32(0.5) * x * y * y)
    return y


@functools.partial(
    pl.kernel,
    mesh=_sc_mesh,
    out_type=jax.ShapeDtypeStruct((NC, LANES), jnp.float32),
    scratch_types=[
        pltpu.VMEM((BPW,), jnp.int32),
        pltpu.VMEM((3, CHUNK, FEAT_DIM), jnp.float32),
        pltpu.VMEM((NCHUNK, CHUNK, FEAT_DIM), jnp.float32),
        pltpu.VMEM((NS * LANES,), jnp.float32),
        pltpu.VMEM((LANES,), jnp.float32),
        pltpu.VMEM_SHARED((NS * LANES,), jnp.float32),
        [pltpu.SemaphoreType.DMA] * 3,
        [pltpu.SemaphoreType.DMA] * NCHUNK,
    ],
    compiler_params=pltpu.CompilerParams(needs_layout_passes=False),
)
def _sc_center_loss(
    centers_hbm,
    feats_hbm,
    idx_hbm,
    out_hbm,
    idx_v,
    f_v,
    c_v,
    gath_v,
    res_v,
    shared,
    f_sems,
    c_sems,
):
    sid = lax.axis_index("s")
    cc = lax.axis_index("c")
    wid = sid * NC + cc
    base = wid * BPW

    def start_f(k):
        return pltpu.async_copy(
            feats_hbm.at[pl.ds(base + k * CHUNK, CHUNK)], f_v.at[k % 3], f_sems[k % 3]
        )

    fh = [start_f(k) for k in range(3)]
    pltpu.sync_copy(idx_hbm.at[pl.ds(base, BPW)], idx_v)
    ch = [
        pltpu.async_copy(
            centers_hbm.at[idx_v.at[pl.ds(k * CHUNK, CHUNK)]], c_v.at[k], c_sems[k]
        )
        for k in range(NCHUNK)
    ]

    def chunk_compute(k, acc):
        f_ref = f_v.at[k % 3]
        c_ref = c_v.at[k]

        def group(g, acc):
            r0 = g * LANES
            for j in range(LANES):
                r = r0 + j
                z = jnp.zeros((LANES,), jnp.float32)
                sf0, sf1, sc0, sc1, fc0, fc1 = z, z, z, z, z, z
                for e in range(FEAT_DIM // LANES):
                    fv = f_ref[r, pl.ds(e * LANES, LANES)]
                    cv = c_ref[r, pl.ds(e * LANES, LANES)]
                    if e % 2 == 0:
                        sf0 = sf0 + fv * fv
                        sc0 = sc0 + cv * cv
                        fc0 = fc0 + fv * cv
                    else:
                        sf1 = sf1 + fv * fv
                        sc1 = sc1 + cv * cv
                        fc1 = fc1 + fv * cv
                sf = jnp.sum(sf0 + sf1)
                sc = jnp.sum(sc0 + sc1)
                fc = jnp.sum(fc0 + fc1)
                acc = acc + fc * _rsqrt_newton(sf) * _rsqrt_newton(sc)
            return acc

        return lax.fori_loop(0, NGROUP, group, acc)

    acc = jnp.float32(0.0)
    for k in range(NCHUNK):
        fh[k % 3].wait()
        ch[k].wait()
        acc = chunk_compute(k, acc)
        if k + 3 < NCHUNK:
            fh[k % 3] = start_f(k + 3)

    lane0 = lax.iota(jnp.int32, LANES) == 0
    res_v[...] = jnp.where(lane0, jnp.full((LANES,), acc), 0.0)
    pltpu.sync_copy(res_v, shared.at[pl.ds(sid * LANES, LANES)])
    plsc.subcore_barrier()

    @pl.when(sid == 0)
    def _():
        pltpu.sync_copy(shared, gath_v)
        tot = jnp.zeros((LANES,), jnp.float32)
        for t in range(NS):
            tot = tot + gath_v[pl.ds(t * LANES, LANES)]
        total = jnp.sum(tot)
        res_v[...] = jnp.where(lane0, jnp.full((LANES,), total), 0.0)
        pltpu.sync_copy(res_v, out_hbm.at[cc])


def kernel(features, labels, centers):
    idx = labels.astype(jnp.int32)
    partials = _sc_center_loss(centers, features, idx)
    return 1.0 - (partials[0, 0] + partials[1, 0]) / jnp.float32(BATCH)


# speedup vs baseline: 1.1099x; 1.1099x over previous
import functools

import jax
import jax.numpy as jnp
from jax import lax
from jax.experimental import pallas as pl
from jax.experimental.pallas import tpu as pltpu
from jax.experimental.pallas import tpu_sc as plsc

NUM_CLASSES = 100000
FEAT_DIM = 128
BATCH = 16384

NC = 2
NS = 16
LANES = 16
NW = NC * NS
BPW = BATCH // NW
CHUNK = 128
NCHUNK = BPW // CHUNK
NGROUP = CHUNK // LANES

_sc_mesh = plsc.VectorSubcoreMesh(core_axis_name="c", subcore_axis_name="s")


def _rsqrt_newton(x):
    xi = lax.bitcast_convert_type(x, jnp.int32)
    yi = jnp.int32(0x5F3759DF) - (xi >> 1)
    y = lax.bitcast_convert_type(yi, jnp.float32)
    for _ in range(3):
        y = y * (jnp.float32(1.5) - jnp.float32(0.5) * x * y * y)
    return y


@functools.partial(
    pl.kernel,
    mesh=_sc_mesh,
    out_type=jax.ShapeDtypeStruct((NC, LANES), jnp.float32),
    scratch_types=[
        pltpu.VMEM((BPW,), jnp.int32),
        pltpu.VMEM((2, CHUNK, FEAT_DIM), jnp.float32),
        pltpu.VMEM((2, CHUNK, FEAT_DIM), jnp.float32),
        pltpu.VMEM((NS * LANES,), jnp.float32),
        pltpu.VMEM((LANES,), jnp.float32),
        pltpu.VMEM_SHARED((NS * LANES,), jnp.float32),
        [pltpu.SemaphoreType.DMA] * 2,
        [pltpu.SemaphoreType.DMA] * 2,
    ],
    compiler_params=pltpu.CompilerParams(needs_layout_passes=False),
)
def _sc_center_loss(
    centers_hbm,
    feats_hbm,
    idx_hbm,
    out_hbm,
    idx_v,
    f_v,
    c_v,
    gath_v,
    res_v,
    shared,
    f_sems,
    c_sems,
):
    sid = lax.axis_index("s")
    cc = lax.axis_index("c")
    wid = sid * NC + cc
    base = wid * BPW

    def start_f(k):
        return pltpu.async_copy(
            feats_hbm.at[pl.ds(base + k * CHUNK, CHUNK)], f_v.at[k % 2], f_sems[k % 2]
        )

    def start_c(k):
        return pltpu.async_copy(
            centers_hbm.at[idx_v.at[pl.ds(k * CHUNK, CHUNK)]], c_v.at[k % 2], c_sems[k % 2]
        )

    fh = [start_f(0), start_f(1)]
    pltpu.sync_copy(idx_hbm.at[pl.ds(base, BPW)], idx_v)
    ch = [start_c(0), start_c(1)]

    def chunk_compute(k, acc):
        f_ref = f_v.at[k % 2]
        c_ref = c_v.at[k % 2]

        def group(g, acc):
            r0 = g * LANES
            for j in range(LANES):
                r = r0 + j
                z = jnp.zeros((LANES,), jnp.float32)
                sf0, sf1, sc0, sc1, fc0, fc1 = z, z, z, z, z, z
                for e in range(FEAT_DIM // LANES):
                    fv = f_ref[r, pl.ds(e * LANES, LANES)]
                    cv = c_ref[r, pl.ds(e * LANES, LANES)]
                    if e % 2 == 0:
                        sf0 = sf0 + fv * fv
                        sc0 = sc0 + cv * cv
                        fc0 = fc0 + fv * cv
                    else:
                        sf1 = sf1 + fv * fv
                        sc1 = sc1 + cv * cv
                        fc1 = fc1 + fv * cv
                sf = jnp.sum(sf0 + sf1)
                sc = jnp.sum(sc0 + sc1)
                fc = jnp.sum(fc0 + fc1)
                acc = acc + fc * _rsqrt_newton(sf * sc)
            return acc

        return lax.fori_loop(0, NGROUP, group, acc)

    acc = jnp.float32(0.0)
    for k in range(NCHUNK):
        fh[k % 2].wait()
        ch[k % 2].wait()
        acc = chunk_compute(k, acc)
        if k + 2 < NCHUNK:
            fh[k % 2] = start_f(k + 2)
            ch[k % 2] = start_c(k + 2)

    lane0 = lax.iota(jnp.int32, LANES) == 0
    res_v[...] = jnp.where(lane0, jnp.full((LANES,), acc), 0.0)
    pltpu.sync_copy(res_v, shared.at[pl.ds(sid * LANES, LANES)])
    plsc.subcore_barrier()

    @pl.when(sid == 0)
    def _():
        pltpu.sync_copy(shared, gath_v)
        tot = jnp.zeros((LANES,), jnp.float32)
        for t in range(NS):
            tot = tot + gath_v[pl.ds(t * LANES, LANES)]
        total = jnp.sum(tot)
        res_v[...] = jnp.where(lane0, jnp.full((LANES,), total), 0.0)
        pltpu.sync_copy(res_v, out_hbm.at[cc])


def kernel(features, labels, centers):
    idx = labels.astype(jnp.int32)
    partials = _sc_center_loss(centers, features, idx)
    return 1.0 - (partials[0, 0] + partials[1, 0]) / jnp.float32(BATCH)
